# baseline (speedup 1.0000x reference)
import functools

import jax
import jax.numpy as jnp
from jax import lax
from jax.experimental import pallas as pl
from jax.experimental.pallas import tpu as pltpu
from jax.experimental.pallas import tpu_sc as plsc

HIDDEN = 768
NUM_EXPERTS = 64

_TOKEN_TILE = 1024
_SIGN = -2147483648

_NC = 2
_NS = 16
_L = 16
_NW = _NC * _NS


def _matmul_kernel(x_ref, w_ref, key_ref, rl_ref):
    lt = lax.dot_general(
        w_ref[...], x_ref[...],
        dimension_numbers=(((1,), (1,)), ((), ())),
        preferred_element_type=jnp.float32,
    )
    rl_ref[...] = lt.T
    bits = lax.bitcast_convert_type(lt, jnp.int32)
    key_ref[...] = jnp.where(bits >= 0, bits | jnp.int32(_SIGN), ~bits)


def _rank_scan(hist, lanes, kk):
    def step(i, carry):
        rt, bstar, rem = carry
        g = jnp.int32(15) - i
        tv = jnp.zeros((_L,), jnp.int32)
        base = (g * _L + lanes) * _L
        for p in range(_L):
            tv = tv + plsc.load_gather(hist, [base + p])
        incl = plsc.cumsum(tv)
        tot = jnp.sum(tv)
        cnt_gt = rt + (tot - incl)
        m = (cnt_gt < kk) & (cnt_gt + tv >= kk)
        bstar = jnp.maximum(bstar, jnp.max(jnp.where(m, g * _L + lanes, -1)))
        rem = jnp.maximum(rem, jnp.max(jnp.where(m, kk - cnt_gt, -1)))
        return rt + tot, bstar, rem

    init = (jnp.int32(0), jnp.int32(-1), jnp.int32(-1))
    _, bstar, rem = lax.fori_loop(0, 16, step, init)
    return bstar, rem


def _zero_hist(hist):
    def z(i, _):
        hist[pl.ds(i * _L, _L)] = jnp.zeros((_L,), jnp.int32)
        return 0

    lax.fori_loop(0, 256, z, 0)


def _make_sc_threshold(n_tokens, k, interpret=False):
    N = n_tokens
    NV = N // _L
    mesh = plsc.VectorSubcoreMesh(core_axis_name="c", subcore_axis_name="s",
                                  num_cores=_NC, num_subcores=_NS)

    @functools.partial(
        pl.kernel,
        interpret=interpret,
        out_type=jax.ShapeDtypeStruct((_NW, _L), jnp.int32),
        mesh=mesh,
        compiler_params=pltpu.CompilerParams(needs_layout_passes=False),
        scratch_types=[
            pltpu.VMEM((N,), jnp.int32),
            pltpu.VMEM((N,), jnp.int32),
            pltpu.VMEM((N,), jnp.int32),
            pltpu.VMEM((256 * _L,), jnp.int32),
            pltpu.VMEM((_L,), jnp.int32),
        ],
    )
    def sc_threshold(keys_hbm, out_hbm, row, cand_v, cand_i, hist, res):
        cid = lax.axis_index("c")
        sid = lax.axis_index("s")
        wid = sid * _NC + cid
        lanes = lax.iota(jnp.int32, _L)
        ones = jnp.ones((_L,), jnp.int32)
        res_vec = jnp.zeros((_L,), jnp.int32)

        for j in range(2):
            e = wid * 2 + j
            pltpu.sync_copy(keys_hbm.at[e], row)

            _zero_hist(hist)

            def h1(i, _):
                u = row[pl.ds(i * _L, _L)]
                b = lax.shift_right_logical(u, 24)
                pos = b * _L + lanes
                cur = plsc.load_gather(hist, [pos])
                plsc.store_scatter(hist, [pos], cur + ones)
                return 0

            lax.fori_loop(0, NV, h1, 0)
            b1, rem = _rank_scan(hist, lanes, jnp.int32(k))

            def c2(i, off):
                u = row[pl.ds(i * _L, _L)]
                m = lax.shift_right_logical(u, 24) == b1
                mi = m.astype(jnp.int32)
                pos = off + plsc.cumsum(mi) - 1
                plsc.store_scatter(cand_v, [pos], u, mask=m)
                plsc.store_scatter(cand_i, [pos], i * _L + lanes, mask=m)
                return off + plsc.all_reduce_population_count(m)

            offs = lax.fori_loop(0, NV, c2, jnp.zeros((_L,), jnp.int32))
            cand_n = jnp.max(offs)
            nv_c = lax.div(cand_n + (_L - 1), jnp.int32(_L))

            prefix = b1
            for sh in (16, 8, 0):
                _zero_hist(hist)

                def hl(i, _, sh=sh, prefix=prefix):
                    g = i * _L + lanes
                    u = cand_v[pl.ds(i * _L, _L)]
                    m = (g < cand_n) & (
                        lax.shift_right_logical(u, sh + 8) == prefix)
                    b = jnp.bitwise_and(lax.shift_right_logical(u, sh), 255)
                    pos = b * _L + lanes
                    cur = plsc.load_gather(hist, [pos])
                    plsc.store_scatter(hist, [pos], cur + ones, mask=m)
                    return 0

                lax.fori_loop(0, nv_c, hl, 0)
                blvl, rem = _rank_scan(hist, lanes, rem)
                prefix = lax.shift_left(prefix, 8) | blvl

            kth = prefix
            need = rem

            def s4(i, carry):
                cum, ans = carry
                g = i * _L + lanes
                u = cand_v[pl.ds(i * _L, _L)]
                m = (u == kth) & (g < cand_n)
                incl = plsc.cumsum(m.astype(jnp.int32))
                hit = m & (cum + incl == need)
                iv = cand_i[pl.ds(i * _L, _L)]
                ans = jnp.maximum(ans, jnp.max(jnp.where(hit, iv, -1)))
                return cum + plsc.all_reduce_population_count(m), ans

            _, idx_thr = lax.fori_loop(
                0, nv_c, s4, (jnp.zeros((_L,), jnp.int32), jnp.int32(-1)))

            okey_thr = kth ^ jnp.int32(_SIGN)
            res_vec = jnp.where(lanes == 2 * j, okey_thr, res_vec)
            res_vec = jnp.where(lanes == 2 * j + 1, idx_thr, res_vec)

        res[...] = res_vec
        pltpu.sync_copy(res, out_hbm.at[wid])

    return sc_threshold


def _assign_kernel(key_ref, thr_ref, idx_ref, score_ref, eidx_ref,
                   counts_ref, *, n_tokens):
    E = NUM_EXPERTS
    N = n_tokens
    SIGN = jnp.int32(_SIGN)
    okey = key_ref[...] ^ SIGN
    oK = thr_ref[...]
    ithr = idx_ref[...]
    tok = lax.broadcasted_iota(jnp.int32, (E, N), 1)
    mask = (okey > oK) | ((okey == oK) & (tok <= ithr))

    bsel = jnp.max(jnp.where(mask, okey, SIGN), axis=0, keepdims=True)
    ball = jnp.max(okey, axis=0, keepdims=True)
    anysel = bsel != SIGN
    best = jnp.where(anysel, bsel, ball)
    eidx = lax.broadcasted_iota(jnp.int32, (E, N), 0)
    expert = jnp.min(
        jnp.where((okey == best) & (mask | ~anysel), eidx, jnp.int32(E)),
        axis=0, keepdims=True,
    )

    bb = jnp.where(best >= 0, best, ~(best ^ SIGN))
    score_ref[...] = lax.bitcast_convert_type(bb, jnp.float32)
    eidx_ref[...] = expert
    counts_ref[...] = jnp.sum(
        (expert == lax.broadcasted_iota(jnp.int32, (E, N), 0))
        .astype(jnp.float32),
        axis=1, keepdims=True,
    )


def _run(x, W, interpret=False):
    B, S, H = x.shape
    E = W.shape[0]
    N = B * S
    k = max(1, min(N // E, N))
    x_flat = x.reshape(N, H)
    n_tiles = N // _TOKEN_TILE

    keys, router_logits = pl.pallas_call(
        _matmul_kernel,
        grid=(n_tiles,),
        in_specs=[
            pl.BlockSpec((_TOKEN_TILE, H), lambda i: (i, 0)),
            pl.BlockSpec((E, H), lambda i: (0, 0)),
        ],
        out_specs=[
            pl.BlockSpec((E, _TOKEN_TILE), lambda i: (0, i)),
            pl.BlockSpec((_TOKEN_TILE, E), lambda i: (i, 0)),
        ],
        out_shape=[
            jax.ShapeDtypeStruct((E, N), jnp.int32),
            jax.ShapeDtypeStruct((N, E), jnp.float32),
        ],
        interpret=interpret,
    )(x_flat, W)

    thr_raw = _make_sc_threshold(N, k, interpret)(keys)
    pairs = thr_raw[:, :4].reshape(_NW * 2, 2)
    oK = pairs[:, 0].reshape(E, 1)
    idx_thr = pairs[:, 1].reshape(E, 1)

    score, expert, counts = pl.pallas_call(
        functools.partial(_assign_kernel, n_tokens=N),
        out_shape=[
            jax.ShapeDtypeStruct((1, N), jnp.float32),
            jax.ShapeDtypeStruct((1, N), jnp.int32),
            jax.ShapeDtypeStruct((E, 1), jnp.float32),
        ],
        interpret=interpret,
    )(keys, oK, idx_thr)

    routing_weights = score.reshape(B, S)
    expert_indices = expert.reshape(B, S)
    expert_counts = counts.reshape(E)
    aux_loss = jnp.float32(0.0)
    return routing_weights, expert_indices, router_logits, aux_loss, expert_counts


def kernel(x, W):
    return _run(x, W, interpret=False)


# speedup vs baseline: 1.0757x; 1.0757x over previous
import functools

import jax
import jax.numpy as jnp
from jax import lax
from jax.experimental import pallas as pl
from jax.experimental.pallas import tpu as pltpu
from jax.experimental.pallas import tpu_sc as plsc

HIDDEN = 768
NUM_EXPERTS = 64

_TOKEN_TILE = 1024
_SIGN = -2147483648

_NC = 2
_NS = 16
_L = 16
_NW = _NC * _NS


def _matmul_kernel(x_ref, w_ref, key_ref, rl_ref):
    lt = lax.dot_general(
        w_ref[...], x_ref[...],
        dimension_numbers=(((1,), (1,)), ((), ())),
        preferred_element_type=jnp.float32,
    )
    rl_ref[...] = lt.T
    bits = lax.bitcast_convert_type(lt, jnp.int32)
    key_ref[...] = jnp.where(bits >= 0, bits | jnp.int32(_SIGN), ~bits)


def _rank_scan(hist, lanes, kk):
    def step(i, carry):
        rt, bstar, rem = carry
        g = jnp.int32(15) - i
        tv = jnp.zeros((_L,), jnp.int32)
        base = (g * _L + lanes) * _L
        for p in range(_L):
            tv = tv + plsc.load_gather(hist, [base + p])
        incl = plsc.cumsum(tv)
        tot = jnp.sum(tv)
        cnt_gt = rt + (tot - incl)
        m = (cnt_gt < kk) & (cnt_gt + tv >= kk)
        bstar = jnp.maximum(bstar, jnp.max(jnp.where(m, g * _L + lanes, -1)))
        rem = jnp.maximum(rem, jnp.max(jnp.where(m, kk - cnt_gt, -1)))
        return rt + tot, bstar, rem

    init = (jnp.int32(0), jnp.int32(-1), jnp.int32(-1))
    _, bstar, rem = lax.fori_loop(0, 16, step, init)
    return bstar, rem


def _zero_hist(hist):
    def z(i, _):
        hist[pl.ds(i * _L, _L)] = jnp.zeros((_L,), jnp.int32)
        return 0

    lax.fori_loop(0, 256, z, 0)


def _make_sc_threshold(n_tokens, k, interpret=False):
    N = n_tokens
    NV = N // _L
    mesh = plsc.VectorSubcoreMesh(core_axis_name="c", subcore_axis_name="s",
                                  num_cores=_NC, num_subcores=_NS)

    @functools.partial(
        pl.kernel,
        interpret=interpret,
        out_type=jax.ShapeDtypeStruct((_NW, _L), jnp.int32),
        mesh=mesh,
        compiler_params=pltpu.CompilerParams(needs_layout_passes=False),
        scratch_types=[
            pltpu.VMEM((N,), jnp.int32),
            pltpu.VMEM((N,), jnp.int32),
            pltpu.VMEM((N,), jnp.int32),
            pltpu.VMEM((256 * _L,), jnp.int32),
            pltpu.VMEM((_L,), jnp.int32),
        ],
    )
    def sc_threshold(keys_hbm, out_hbm, row, cand_v, cand_i, hist, res):
        cid = lax.axis_index("c")
        sid = lax.axis_index("s")
        wid = sid * _NC + cid
        lanes = lax.iota(jnp.int32, _L)
        ones = jnp.ones((_L,), jnp.int32)
        res_vec = jnp.zeros((_L,), jnp.int32)

        for j in range(2):
            e = wid * 2 + j
            pltpu.sync_copy(keys_hbm.at[e], row)

            _zero_hist(hist)

            def h1(i, _):
                u = row[pl.ds(i * _L, _L)]
                b = lax.shift_right_logical(u, 24)
                plsc.addupdate_scatter(hist, [b * _L + lanes], ones)
                return 0

            lax.fori_loop(0, NV, h1, 0)
            b1, rem = _rank_scan(hist, lanes, jnp.int32(k))

            def c2(i, off):
                u = row[pl.ds(i * _L, _L)]
                m = lax.shift_right_logical(u, 24) == b1
                mi = m.astype(jnp.int32)
                pos = off + plsc.cumsum(mi) - 1
                plsc.store_scatter(cand_v, [pos], u, mask=m)
                plsc.store_scatter(cand_i, [pos], i * _L + lanes, mask=m)
                return off + plsc.all_reduce_population_count(m)

            offs = lax.fori_loop(0, NV, c2, jnp.zeros((_L,), jnp.int32))
            cand_n = jnp.max(offs)
            nv_c = lax.div(cand_n + (_L - 1), jnp.int32(_L))

            prefix = b1
            for sh in (16, 8, 0):
                _zero_hist(hist)

                def hl(i, _, sh=sh, prefix=prefix):
                    g = i * _L + lanes
                    u = cand_v[pl.ds(i * _L, _L)]
                    m = (g < cand_n) & (
                        lax.shift_right_logical(u, sh + 8) == prefix)
                    b = jnp.bitwise_and(lax.shift_right_logical(u, sh), 255)
                    plsc.addupdate_scatter(hist, [b * _L + lanes], ones,
                                           mask=m)
                    return 0

                lax.fori_loop(0, nv_c, hl, 0)
                blvl, rem = _rank_scan(hist, lanes, rem)
                prefix = lax.shift_left(prefix, 8) | blvl

            kth = prefix
            need = rem

            def s4(i, carry):
                cum, ans = carry
                g = i * _L + lanes
                u = cand_v[pl.ds(i * _L, _L)]
                m = (u == kth) & (g < cand_n)
                incl = plsc.cumsum(m.astype(jnp.int32))
                hit = m & (cum + incl == need)
                iv = cand_i[pl.ds(i * _L, _L)]
                ans = jnp.maximum(ans, jnp.max(jnp.where(hit, iv, -1)))
                return cum + plsc.all_reduce_population_count(m), ans

            _, idx_thr = lax.fori_loop(
                0, nv_c, s4, (jnp.zeros((_L,), jnp.int32), jnp.int32(-1)))

            okey_thr = kth ^ jnp.int32(_SIGN)
            res_vec = jnp.where(lanes == 2 * j, okey_thr, res_vec)
            res_vec = jnp.where(lanes == 2 * j + 1, idx_thr, res_vec)

        res[...] = res_vec
        pltpu.sync_copy(res, out_hbm.at[wid])

    return sc_threshold


def _assign_kernel(key_ref, thr_ref, idx_ref, score_ref, eidx_ref,
                   counts_ref, *, n_tokens):
    E = NUM_EXPERTS
    N = n_tokens
    SIGN = jnp.int32(_SIGN)
    okey = key_ref[...] ^ SIGN
    oK = thr_ref[...]
    ithr = idx_ref[...]
    tok = lax.broadcasted_iota(jnp.int32, (E, N), 1)
    mask = (okey > oK) | ((okey == oK) & (tok <= ithr))

    bsel = jnp.max(jnp.where(mask, okey, SIGN), axis=0, keepdims=True)
    ball = jnp.max(okey, axis=0, keepdims=True)
    anysel = bsel != SIGN
    best = jnp.where(anysel, bsel, ball)
    eidx = lax.broadcasted_iota(jnp.int32, (E, N), 0)
    expert = jnp.min(
        jnp.where((okey == best) & (mask | ~anysel), eidx, jnp.int32(E)),
        axis=0, keepdims=True,
    )

    bb = jnp.where(best >= 0, best, ~(best ^ SIGN))
    score_ref[...] = lax.bitcast_convert_type(bb, jnp.float32)
    eidx_ref[...] = expert
    counts_ref[...] = jnp.sum(
        (expert == lax.broadcasted_iota(jnp.int32, (E, N), 0))
        .astype(jnp.float32),
        axis=1, keepdims=True,
    )


def _run(x, W, interpret=False):
    B, S, H = x.shape
    E = W.shape[0]
    N = B * S
    k = max(1, min(N // E, N))
    x_flat = x.reshape(N, H)
    n_tiles = N // _TOKEN_TILE

    keys, router_logits = pl.pallas_call(
        _matmul_kernel,
        grid=(n_tiles,),
        in_specs=[
            pl.BlockSpec((_TOKEN_TILE, H), lambda i: (i, 0)),
            pl.BlockSpec((E, H), lambda i: (0, 0)),
        ],
        out_specs=[
            pl.BlockSpec((E, _TOKEN_TILE), lambda i: (0, i)),
            pl.BlockSpec((_TOKEN_TILE, E), lambda i: (i, 0)),
        ],
        out_shape=[
            jax.ShapeDtypeStruct((E, N), jnp.int32),
            jax.ShapeDtypeStruct((N, E), jnp.float32),
        ],
        interpret=interpret,
    )(x_flat, W)

    thr_raw = _make_sc_threshold(N, k, interpret)(keys)
    pairs = thr_raw[:, :4].reshape(_NW * 2, 2)
    oK = pairs[:, 0].reshape(E, 1)
    idx_thr = pairs[:, 1].reshape(E, 1)

    score, expert, counts = pl.pallas_call(
        functools.partial(_assign_kernel, n_tokens=N),
        out_shape=[
            jax.ShapeDtypeStruct((1, N), jnp.float32),
            jax.ShapeDtypeStruct((1, N), jnp.int32),
            jax.ShapeDtypeStruct((E, 1), jnp.float32),
        ],
        interpret=interpret,
    )(keys, oK, idx_thr)

    routing_weights = score.reshape(B, S)
    expert_indices = expert.reshape(B, S)
    expert_counts = counts.reshape(E)
    aux_loss = jnp.float32(0.0)
    return routing_weights, expert_indices, router_logits, aux_loss, expert_counts


def kernel(x, W):
    return _run(x, W, interpret=False)


# speedup vs baseline: 1.1500x; 1.0691x over previous
import functools

import jax
import jax.numpy as jnp
from jax import lax
from jax.experimental import pallas as pl
from jax.experimental.pallas import tpu as pltpu
from jax.experimental.pallas import tpu_sc as plsc

HIDDEN = 768
NUM_EXPERTS = 64

_TOKEN_TILE = 1024
_SIGN = -2147483648

_NC = 2
_NS = 16
_L = 16
_NW = _NC * _NS


def _matmul_kernel(x_ref, w_ref, key_ref, rl_ref):
    lt = lax.dot_general(
        w_ref[...], x_ref[...],
        dimension_numbers=(((1,), (1,)), ((), ())),
        preferred_element_type=jnp.float32,
    )
    rl_ref[...] = lt.T
    bits = lax.bitcast_convert_type(lt, jnp.int32)
    key_ref[...] = jnp.where(bits >= 0, bits | jnp.int32(_SIGN), ~bits)


def _rank_scan(hist, lanes, kk):
    def step(i, carry):
        rt, bstar, rem = carry
        g = jnp.int32(15) - i
        tv = jnp.zeros((_L,), jnp.int32)
        base = (g * _L + lanes) * _L
        for p in range(_L):
            tv = tv + plsc.load_gather(hist, [base + p])
        incl = plsc.cumsum(tv)
        tot = jnp.sum(tv)
        cnt_gt = rt + (tot - incl)
        m = (cnt_gt < kk) & (cnt_gt + tv >= kk)
        bstar = jnp.maximum(bstar, jnp.max(jnp.where(m, g * _L + lanes, -1)))
        rem = jnp.maximum(rem, jnp.max(jnp.where(m, kk - cnt_gt, -1)))
        return rt + tot, bstar, rem

    init = (jnp.int32(0), jnp.int32(-1), jnp.int32(-1))
    _, bstar, rem = lax.fori_loop(0, 16, step, init)
    return bstar, rem


def _zero_hist(hist):
    zv = jnp.zeros((_L,), jnp.int32)

    def z(i, _):
        for p in range(8):
            hist[pl.ds((i * 8 + p) * _L, _L)] = zv
        return 0

    lax.fori_loop(0, 32, z, 0)


def _make_sc_threshold(n_tokens, k, interpret=False):
    N = n_tokens
    NV = N // _L
    mesh = plsc.VectorSubcoreMesh(core_axis_name="c", subcore_axis_name="s",
                                  num_cores=_NC, num_subcores=_NS)

    @functools.partial(
        pl.kernel,
        interpret=interpret,
        out_type=jax.ShapeDtypeStruct((_NW, _L), jnp.int32),
        mesh=mesh,
        compiler_params=pltpu.CompilerParams(needs_layout_passes=False),
        scratch_types=[
            pltpu.VMEM((N,), jnp.int32),
            pltpu.VMEM((N,), jnp.int32),
            pltpu.VMEM((N,), jnp.int32),
            pltpu.VMEM((256 * _L,), jnp.int32),
            pltpu.VMEM((_L,), jnp.int32),
        ],
    )
    def sc_threshold(keys_hbm, out_hbm, row, cand_v, cand_i, hist, res):
        cid = lax.axis_index("c")
        sid = lax.axis_index("s")
        wid = sid * _NC + cid
        lanes = lax.iota(jnp.int32, _L)
        ones = jnp.ones((_L,), jnp.int32)
        res_vec = jnp.zeros((_L,), jnp.int32)

        for j in range(2):
            e = wid * 2 + j
            pltpu.sync_copy(keys_hbm.at[e], row)

            _zero_hist(hist)

            def h1(i, _):
                for p in range(8):
                    u = row[pl.ds((i * 8 + p) * _L, _L)]
                    b = lax.shift_right_logical(u, 24)
                    plsc.addupdate_scatter(hist, [b * _L + lanes], ones)
                return 0

            lax.fori_loop(0, NV // 8, h1, 0)
            b1, rem = _rank_scan(hist, lanes, jnp.int32(k))

            def c2(i, off):
                for p in range(8):
                    g = (i * 8 + p) * _L
                    u = row[pl.ds(g, _L)]
                    m = lax.shift_right_logical(u, 24) == b1
                    mi = m.astype(jnp.int32)
                    pos = off + plsc.cumsum(mi) - 1
                    plsc.store_scatter(cand_v, [pos], u, mask=m)
                    plsc.store_scatter(cand_i, [pos], g + lanes, mask=m)
                    off = off + plsc.all_reduce_population_count(m)
                return off

            offs = lax.fori_loop(0, NV // 8, c2, jnp.zeros((_L,), jnp.int32))
            cand_n = jnp.max(offs)
            nv_c = lax.div(cand_n + (_L - 1), jnp.int32(_L))

            prefix = b1
            for sh in (16, 8, 0):
                _zero_hist(hist)

                def hl(i, _, sh=sh, prefix=prefix):
                    g = i * _L + lanes
                    u = cand_v[pl.ds(i * _L, _L)]
                    m = (g < cand_n) & (
                        lax.shift_right_logical(u, sh + 8) == prefix)
                    b = jnp.bitwise_and(lax.shift_right_logical(u, sh), 255)
                    plsc.addupdate_scatter(hist, [b * _L + lanes], ones,
                                           mask=m)
                    return 0

                lax.fori_loop(0, nv_c, hl, 0)
                blvl, rem = _rank_scan(hist, lanes, rem)
                prefix = lax.shift_left(prefix, 8) | blvl

            kth = prefix
            need = rem

            def s4(i, carry):
                cum, ans = carry
                g = i * _L + lanes
                u = cand_v[pl.ds(i * _L, _L)]
                m = (u == kth) & (g < cand_n)
                incl = plsc.cumsum(m.astype(jnp.int32))
                hit = m & (cum + incl == need)
                iv = cand_i[pl.ds(i * _L, _L)]
                ans = jnp.maximum(ans, jnp.max(jnp.where(hit, iv, -1)))
                return cum + plsc.all_reduce_population_count(m), ans

            _, idx_thr = lax.fori_loop(
                0, nv_c, s4, (jnp.zeros((_L,), jnp.int32), jnp.int32(-1)))

            okey_thr = kth ^ jnp.int32(_SIGN)
            res_vec = jnp.where(lanes == 2 * j, okey_thr, res_vec)
            res_vec = jnp.where(lanes == 2 * j + 1, idx_thr, res_vec)

        res[...] = res_vec
        pltpu.sync_copy(res, out_hbm.at[wid])

    return sc_threshold


def _assign_kernel(key_ref, thr_ref, idx_ref, score_ref, eidx_ref,
                   counts_ref, *, n_tokens):
    E = NUM_EXPERTS
    N = n_tokens
    SIGN = jnp.int32(_SIGN)
    okey = key_ref[...] ^ SIGN
    oK = thr_ref[...]
    ithr = idx_ref[...]
    tok = lax.broadcasted_iota(jnp.int32, (E, N), 1)
    mask = (okey > oK) | ((okey == oK) & (tok <= ithr))

    bsel = jnp.max(jnp.where(mask, okey, SIGN), axis=0, keepdims=True)
    ball = jnp.max(okey, axis=0, keepdims=True)
    anysel = bsel != SIGN
    best = jnp.where(anysel, bsel, ball)
    eidx = lax.broadcasted_iota(jnp.int32, (E, N), 0)
    expert = jnp.min(
        jnp.where((okey == best) & (mask | ~anysel), eidx, jnp.int32(E)),
        axis=0, keepdims=True,
    )

    bb = jnp.where(best >= 0, best, ~(best ^ SIGN))
    score_ref[...] = lax.bitcast_convert_type(bb, jnp.float32)
    eidx_ref[...] = expert
    counts_ref[...] = jnp.sum(
        (expert == lax.broadcasted_iota(jnp.int32, (E, N), 0))
        .astype(jnp.float32),
        axis=1, keepdims=True,
    )


def _run(x, W, interpret=False):
    B, S, H = x.shape
    E = W.shape[0]
    N = B * S
    k = max(1, min(N // E, N))
    x_flat = x.reshape(N, H)
    n_tiles = N // _TOKEN_TILE

    keys, router_logits = pl.pallas_call(
        _matmul_kernel,
        grid=(n_tiles,),
        in_specs=[
            pl.BlockSpec((_TOKEN_TILE, H), lambda i: (i, 0)),
            pl.BlockSpec((E, H), lambda i: (0, 0)),
        ],
        out_specs=[
            pl.BlockSpec((E, _TOKEN_TILE), lambda i: (0, i)),
            pl.BlockSpec((_TOKEN_TILE, E), lambda i: (i, 0)),
        ],
        out_shape=[
            jax.ShapeDtypeStruct((E, N), jnp.int32),
            jax.ShapeDtypeStruct((N, E), jnp.float32),
        ],
        interpret=interpret,
    )(x_flat, W)

    thr_raw = _make_sc_threshold(N, k, interpret)(keys)
    pairs = thr_raw[:, :4].reshape(_NW * 2, 2)
    oK = pairs[:, 0].reshape(E, 1)
    idx_thr = pairs[:, 1].reshape(E, 1)

    score, expert, counts = pl.pallas_call(
        functools.partial(_assign_kernel, n_tokens=N),
        out_shape=[
            jax.ShapeDtypeStruct((1, N), jnp.float32),
            jax.ShapeDtypeStruct((1, N), jnp.int32),
            jax.ShapeDtypeStruct((E, 1), jnp.float32),
        ],
        interpret=interpret,
    )(keys, oK, idx_thr)

    routing_weights = score.reshape(B, S)
    expert_indices = expert.reshape(B, S)
    expert_counts = counts.reshape(E)
    aux_loss = jnp.float32(0.0)
    return routing_weights, expert_indices, router_logits, aux_loss, expert_counts


def kernel(x, W):
    return _run(x, W, interpret=False)
